# Initial kernel scaffold; baseline (speedup 1.0000x reference)
import jax
import jax.numpy as jnp
from jax.experimental import pallas as pl


def kernel(all_triples, hist_tails, hist_len, entity_embeddings, relation_embeddings, W_ih, W_hh, b_ih, b_hh):
    raise NotImplementedError("write your pallas kernel here")


# speedup vs baseline: 15.8420x; 15.8420x over previous
import functools

import jax
import jax.numpy as jnp
from jax import lax
from jax.experimental import pallas as pl
from jax.experimental.pallas import tpu as pltpu
from jax.experimental.pallas import tpu_sc as plsc

_NC = 2
_NS = 16
_NW = _NC * _NS
_CHUNK = 128


def _stream_gather(table, idx_hbm, out_hbm, wid, nch, idx_v, bufs, gsems, wsems):
    pltpu.sync_copy(idx_hbm.at[wid], idx_v)
    base = wid * nch * _CHUNK

    def _gather(c, k):
        return pltpu.make_async_copy(table.at[idx_v.at[c]], bufs[k], gsems[k])

    def _wb(c, k):
        dst = out_hbm.at[pl.ds(base + c * _CHUNK, _CHUNK)]
        return pltpu.make_async_copy(bufs[k], dst, wsems[k])

    for k in range(2):
        _gather(k, k).start()

    def outer(i2, carry):
        for k in range(2):
            c = i2 * 2 + k
            _gather(c, k).wait()
            _wb(c, k).start()

            @pl.when(c + 2 < nch)
            def _():
                _wb(c, k).wait()
                _gather(c + 2, k).start()

        return carry

    lax.fori_loop(0, nch // 2, outer, 0, unroll=False)

    for k in range(2):
        _wb(nch - 2 + k, k).wait()


def _sc_gather_body(ent_hbm, rel_hbm, tidx_hbm, sidx_hbm, ridx_hbm,
                    tails_out, s_out, r_out,
                    iv_t, iv_s, iv_r, buf0, buf1, g0, g1, w0, w1):
    wid = lax.axis_index("s") * _NC + lax.axis_index("c")
    bufs = (buf0, buf1)
    gsems = (g0, g1)
    wsems = (w0, w1)
    nch_t = iv_t.shape[0]
    nch_s = iv_s.shape[0]
    nch_r = iv_r.shape[0]
    _stream_gather(ent_hbm, tidx_hbm, tails_out, wid, nch_t, iv_t, bufs, gsems, wsems)
    _stream_gather(ent_hbm, sidx_hbm, s_out, wid, nch_s, iv_s, bufs, gsems, wsems)
    _stream_gather(rel_hbm, ridx_hbm, r_out, wid, nch_r, iv_r, bufs, gsems, wsems)


def _sc_gather(entity_embeddings, relation_embeddings, tidx, sidx, ridx, H):
    nch_t = tidx.shape[1]
    nch_s = sidx.shape[1]
    nch_r = ridx.shape[1]
    mesh = plsc.VectorSubcoreMesh(core_axis_name="c", subcore_axis_name="s",
                                  num_cores=_NC, num_subcores=_NS)
    f32 = jnp.float32
    kern = pl.kernel(
        _sc_gather_body,
        out_type=(
            jax.ShapeDtypeStruct((_NW * nch_t * _CHUNK, H), f32),
            jax.ShapeDtypeStruct((_NW * nch_s * _CHUNK, H), f32),
            jax.ShapeDtypeStruct((_NW * nch_r * _CHUNK, H), f32),
        ),
        mesh=mesh,
        scratch_types=[
            pltpu.VMEM((nch_t, _CHUNK), jnp.int32),
            pltpu.VMEM((nch_s, _CHUNK), jnp.int32),
            pltpu.VMEM((nch_r, _CHUNK), jnp.int32),
            pltpu.VMEM((_CHUNK, H), f32),
            pltpu.VMEM((_CHUNK, H), f32),
            pltpu.SemaphoreType.DMA,
            pltpu.SemaphoreType.DMA,
            pltpu.SemaphoreType.DMA,
            pltpu.SemaphoreType.DMA,
        ],
    )
    return kern(entity_embeddings, relation_embeddings, tidx, sidx, ridx)


def _gru_body(tails_ref, s_ref, r_ref, hl_ref, wt_ref, whh_ref, bih_ref,
              bhh_ref, out_ref, gib_ref):
    t = pl.program_id(1)
    H = out_ref.shape[1]
    f32 = jnp.float32

    @pl.when(t == 0)
    def _():
        s = s_ref[...]
        r = r_ref[...]
        gib_ref[...] = (
            jnp.dot(s, wt_ref[0:H, :], preferred_element_type=f32)
            + jnp.dot(r, wt_ref[H:2 * H, :], preferred_element_type=f32)
            + bih_ref[...]
        )
        out_ref[...] = jnp.zeros_like(out_ref)

    h = out_ref[...]
    x_t = tails_ref[0]
    gi = gib_ref[...] + jnp.dot(x_t, wt_ref[2 * H:3 * H, :],
                                preferred_element_type=f32)
    gh = jnp.dot(h, whh_ref[...], preferred_element_type=f32) + bhh_ref[...]
    i_r, i_z, i_n = gi[:, :H], gi[:, H:2 * H], gi[:, 2 * H:]
    h_r, h_z, h_n = gh[:, :H], gh[:, H:2 * H], gh[:, 2 * H:]
    rg = jax.nn.sigmoid(i_r + h_r)
    z = jax.nn.sigmoid(i_z + h_z)
    n = jnp.tanh(i_n + rg * h_n)
    h_new = (1.0 - z) * n + z * h
    m = hl_ref[0] > t
    out_ref[...] = jnp.where(m, h_new, h)


def _gru(tails, s_rows, r_rows, hist_len, W_ih, W_hh, b_ih, b_hh, BB):
    T, B, H = tails.shape
    NB = B // BB
    wt = W_ih.T.astype(jnp.float32)
    whh = W_hh.T.astype(jnp.float32)
    bih = b_ih.reshape(1, 3 * H).astype(jnp.float32)
    bhh = b_hh.reshape(1, 3 * H).astype(jnp.float32)
    hl3 = hist_len.astype(jnp.int32).reshape(NB, BB, 1)

    grid = (NB, T)
    return pl.pallas_call(
        _gru_body,
        grid=grid,
        in_specs=[
            pl.BlockSpec((1, BB, H), lambda b, t: (t, b, 0)),
            pl.BlockSpec((BB, H), lambda b, t: (b, 0)),
            pl.BlockSpec((BB, H), lambda b, t: (b, 0)),
            pl.BlockSpec((1, BB, 1), lambda b, t: (b, 0, 0)),
            pl.BlockSpec((3 * H, 3 * H), lambda b, t: (0, 0)),
            pl.BlockSpec((H, 3 * H), lambda b, t: (0, 0)),
            pl.BlockSpec((1, 3 * H), lambda b, t: (0, 0)),
            pl.BlockSpec((1, 3 * H), lambda b, t: (0, 0)),
        ],
        out_specs=pl.BlockSpec((BB, H), lambda b, t: (b, 0)),
        out_shape=jax.ShapeDtypeStruct((B, H), jnp.float32),
        scratch_shapes=[pltpu.VMEM((BB, 3 * H), jnp.float32)],
        compiler_params=pltpu.CompilerParams(
            dimension_semantics=("arbitrary", "arbitrary"),
        ),
    )(tails, s_rows, r_rows, hl3, wt, whh, bih, bhh)


@jax.jit
def kernel(all_triples, hist_tails, hist_len, entity_embeddings,
           relation_embeddings, W_ih, W_hh, b_ih, b_hh):
    B, T = hist_tails.shape
    H = entity_embeddings.shape[1]

    tidx = hist_tails.T.astype(jnp.int32).reshape(_NW, (T * B) // (_NW * _CHUNK), _CHUNK)
    sidx = all_triples[:, 0].astype(jnp.int32).reshape(_NW, B // (_NW * _CHUNK), _CHUNK)
    ridx = all_triples[:, 1].astype(jnp.int32).reshape(_NW, B // (_NW * _CHUNK), _CHUNK)

    tails_flat, s_rows, r_rows = _sc_gather(
        entity_embeddings, relation_embeddings, tidx, sidx, ridx, H)
    tails = tails_flat.reshape(T, B, H)

    return _gru(tails, s_rows, r_rows, hist_len, W_ih, W_hh, b_ih, b_hh, BB=1024)


# speedup vs baseline: 17.7791x; 1.1223x over previous
import functools

import jax
import jax.numpy as jnp
from jax import lax
from jax.experimental import pallas as pl
from jax.experimental.pallas import tpu as pltpu
from jax.experimental.pallas import tpu_sc as plsc

_NC = 2
_NS = 16
_NW = _NC * _NS
_CHUNK = 128


def _stream_gather(table, idx_hbm, out_hbm, wid, nch, idx_v, bufs, gsems, wsems):
    pltpu.sync_copy(idx_hbm.at[wid], idx_v)
    base = wid * nch * _CHUNK

    def _gather(c, k):
        return pltpu.make_async_copy(table.at[idx_v.at[c]], bufs[k], gsems[k])

    def _wb(c, k):
        dst = out_hbm.at[pl.ds(base + c * _CHUNK, _CHUNK)]
        return pltpu.make_async_copy(bufs[k], dst, wsems[k])

    for k in range(2):
        _gather(k, k).start()

    def outer(i2, carry):
        for k in range(2):
            c = i2 * 2 + k
            _gather(c, k).wait()
            _wb(c, k).start()

            @pl.when(c + 2 < nch)
            def _():
                _wb(c, k).wait()
                _gather(c + 2, k).start()

        return carry

    lax.fori_loop(0, nch // 2, outer, 0, unroll=False)

    for k in range(2):
        _wb(nch - 2 + k, k).wait()


def _sc_gather_body(ent_hbm, rel_hbm, tidx_hbm, sidx_hbm, ridx_hbm,
                    tails_out, s_out, r_out,
                    iv_t, iv_s, iv_r, buf0, buf1, g0, g1, w0, w1):
    wid = lax.axis_index("s") * _NC + lax.axis_index("c")
    bufs = (buf0, buf1)
    gsems = (g0, g1)
    wsems = (w0, w1)
    nch_t = iv_t.shape[0]
    nch_s = iv_s.shape[0]
    nch_r = iv_r.shape[0]
    _stream_gather(ent_hbm, tidx_hbm, tails_out, wid, nch_t, iv_t, bufs, gsems, wsems)
    _stream_gather(ent_hbm, sidx_hbm, s_out, wid, nch_s, iv_s, bufs, gsems, wsems)
    _stream_gather(rel_hbm, ridx_hbm, r_out, wid, nch_r, iv_r, bufs, gsems, wsems)


def _sc_gather(entity_embeddings, relation_embeddings, tidx, sidx, ridx, H):
    nch_t = tidx.shape[1]
    nch_s = sidx.shape[1]
    nch_r = ridx.shape[1]
    mesh = plsc.VectorSubcoreMesh(core_axis_name="c", subcore_axis_name="s",
                                  num_cores=_NC, num_subcores=_NS)
    f32 = jnp.float32
    kern = pl.kernel(
        _sc_gather_body,
        out_type=(
            jax.ShapeDtypeStruct((_NW * nch_t * _CHUNK, H), f32),
            jax.ShapeDtypeStruct((_NW * nch_s * _CHUNK, H), f32),
            jax.ShapeDtypeStruct((_NW * nch_r * _CHUNK, H), f32),
        ),
        mesh=mesh,
        scratch_types=[
            pltpu.VMEM((nch_t, _CHUNK), jnp.int32),
            pltpu.VMEM((nch_s, _CHUNK), jnp.int32),
            pltpu.VMEM((nch_r, _CHUNK), jnp.int32),
            pltpu.VMEM((_CHUNK, H), f32),
            pltpu.VMEM((_CHUNK, H), f32),
            pltpu.SemaphoreType.DMA,
            pltpu.SemaphoreType.DMA,
            pltpu.SemaphoreType.DMA,
            pltpu.SemaphoreType.DMA,
        ],
    )
    return kern(entity_embeddings, relation_embeddings, tidx, sidx, ridx)


def _gru_body(tails_ref, s_ref, r_ref, hl_ref, wt_ref, whh_ref, bih_ref,
              bhh_ref, out_ref, gib_ref):
    t = pl.program_id(1)
    H = out_ref.shape[1]
    f32 = jnp.float32
    bf16 = jnp.bfloat16

    @pl.when(t == 0)
    def _():
        s = s_ref[...].astype(bf16)
        r = r_ref[...].astype(bf16)
        gib_ref[...] = (
            jnp.dot(s, wt_ref[0:H, :], preferred_element_type=f32)
            + jnp.dot(r, wt_ref[H:2 * H, :], preferred_element_type=f32)
            + bih_ref[...]
        )
        out_ref[...] = jnp.zeros_like(out_ref)

    h = out_ref[...]
    x_t = tails_ref[0].astype(bf16)
    gi = gib_ref[...] + jnp.dot(x_t, wt_ref[2 * H:3 * H, :],
                                preferred_element_type=f32)
    gh = jnp.dot(h.astype(bf16), whh_ref[...],
                 preferred_element_type=f32) + bhh_ref[...]
    i_r, i_z, i_n = gi[:, :H], gi[:, H:2 * H], gi[:, 2 * H:]
    h_r, h_z, h_n = gh[:, :H], gh[:, H:2 * H], gh[:, 2 * H:]
    rg = jax.nn.sigmoid(i_r + h_r)
    z = jax.nn.sigmoid(i_z + h_z)
    n = jnp.tanh(i_n + rg * h_n)
    h_new = (1.0 - z) * n + z * h
    m = hl_ref[0] > t
    out_ref[...] = jnp.where(m, h_new, h)


def _gru(tails, s_rows, r_rows, hist_len, W_ih, W_hh, b_ih, b_hh, BB):
    T, B, H = tails.shape
    NB = B // BB
    wt = W_ih.T.astype(jnp.bfloat16)
    whh = W_hh.T.astype(jnp.bfloat16)
    bih = b_ih.reshape(1, 3 * H).astype(jnp.float32)
    bhh = b_hh.reshape(1, 3 * H).astype(jnp.float32)
    hl3 = hist_len.astype(jnp.int32).reshape(NB, BB, 1)

    grid = (NB, T)
    return pl.pallas_call(
        _gru_body,
        grid=grid,
        in_specs=[
            pl.BlockSpec((1, BB, H), lambda b, t: (t, b, 0)),
            pl.BlockSpec((BB, H), lambda b, t: (b, 0)),
            pl.BlockSpec((BB, H), lambda b, t: (b, 0)),
            pl.BlockSpec((1, BB, 1), lambda b, t: (b, 0, 0)),
            pl.BlockSpec((3 * H, 3 * H), lambda b, t: (0, 0)),
            pl.BlockSpec((H, 3 * H), lambda b, t: (0, 0)),
            pl.BlockSpec((1, 3 * H), lambda b, t: (0, 0)),
            pl.BlockSpec((1, 3 * H), lambda b, t: (0, 0)),
        ],
        out_specs=pl.BlockSpec((BB, H), lambda b, t: (b, 0)),
        out_shape=jax.ShapeDtypeStruct((B, H), jnp.float32),
        scratch_shapes=[pltpu.VMEM((BB, 3 * H), jnp.float32)],
        compiler_params=pltpu.CompilerParams(
            dimension_semantics=("arbitrary", "arbitrary"),
        ),
    )(tails, s_rows, r_rows, hl3, wt, whh, bih, bhh)


@jax.jit
def kernel(all_triples, hist_tails, hist_len, entity_embeddings,
           relation_embeddings, W_ih, W_hh, b_ih, b_hh):
    B, T = hist_tails.shape
    H = entity_embeddings.shape[1]

    tidx = hist_tails.T.astype(jnp.int32).reshape(_NW, (T * B) // (_NW * _CHUNK), _CHUNK)
    sidx = all_triples[:, 0].astype(jnp.int32).reshape(_NW, B // (_NW * _CHUNK), _CHUNK)
    ridx = all_triples[:, 1].astype(jnp.int32).reshape(_NW, B // (_NW * _CHUNK), _CHUNK)

    tails_flat, s_rows, r_rows = _sc_gather(
        entity_embeddings, relation_embeddings, tidx, sidx, ridx, H)
    tails = tails_flat.reshape(T, B, H)

    return _gru(tails, s_rows, r_rows, hist_len, W_ih, W_hh, b_ih, b_hh, BB=2048)


# speedup vs baseline: 18.5181x; 1.0416x over previous
import functools

import jax
import jax.numpy as jnp
from jax import lax
from jax.experimental import pallas as pl
from jax.experimental.pallas import tpu as pltpu
from jax.experimental.pallas import tpu_sc as plsc

_NC = 2
_NS = 16
_NW = _NC * _NS
_CHUNK = 128


def _stream_gather(table, idx_hbm, out_hbm, wid, nch, idx_v, bufs, gsems, wsems):
    pltpu.sync_copy(idx_hbm.at[wid], idx_v)
    base = wid * nch * _CHUNK

    def _gather(c, k):
        return pltpu.make_async_copy(table.at[idx_v.at[c]], bufs[k], gsems[k])

    def _wb(c, k):
        dst = out_hbm.at[pl.ds(base + c * _CHUNK, _CHUNK)]
        return pltpu.make_async_copy(bufs[k], dst, wsems[k])

    for k in range(2):
        _gather(k, k).start()

    def outer(i2, carry):
        for k in range(2):
            c = i2 * 2 + k
            _gather(c, k).wait()
            _wb(c, k).start()

            @pl.when(c + 2 < nch)
            def _():
                _wb(c, k).wait()
                _gather(c + 2, k).start()

        return carry

    lax.fori_loop(0, nch // 2, outer, 0, unroll=False)

    for k in range(2):
        _wb(nch - 2 + k, k).wait()


def _sc_gather_body(ent_hbm, rel_hbm, tidx_hbm, sidx_hbm, ridx_hbm,
                    tails_out, s_out, r_out,
                    iv_t, iv_s, iv_r, buf0, buf1, g0, g1, w0, w1):
    wid = lax.axis_index("s") * _NC + lax.axis_index("c")
    bufs = (buf0, buf1)
    gsems = (g0, g1)
    wsems = (w0, w1)
    nch_t = iv_t.shape[0]
    nch_s = iv_s.shape[0]
    nch_r = iv_r.shape[0]
    _stream_gather(ent_hbm, tidx_hbm, tails_out, wid, nch_t, iv_t, bufs, gsems, wsems)
    _stream_gather(ent_hbm, sidx_hbm, s_out, wid, nch_s, iv_s, bufs, gsems, wsems)
    _stream_gather(rel_hbm, ridx_hbm, r_out, wid, nch_r, iv_r, bufs, gsems, wsems)


def _sc_gather(entity_embeddings, relation_embeddings, tidx, sidx, ridx, H):
    nch_t = tidx.shape[1]
    nch_s = sidx.shape[1]
    nch_r = ridx.shape[1]
    mesh = plsc.VectorSubcoreMesh(core_axis_name="c", subcore_axis_name="s",
                                  num_cores=_NC, num_subcores=_NS)
    f32 = jnp.float32
    kern = pl.kernel(
        _sc_gather_body,
        out_type=(
            jax.ShapeDtypeStruct((_NW * nch_t * _CHUNK, H), f32),
            jax.ShapeDtypeStruct((_NW * nch_s * _CHUNK, H), f32),
            jax.ShapeDtypeStruct((_NW * nch_r * _CHUNK, H), f32),
        ),
        mesh=mesh,
        scratch_types=[
            pltpu.VMEM((nch_t, _CHUNK), jnp.int32),
            pltpu.VMEM((nch_s, _CHUNK), jnp.int32),
            pltpu.VMEM((nch_r, _CHUNK), jnp.int32),
            pltpu.VMEM((_CHUNK, H), f32),
            pltpu.VMEM((_CHUNK, H), f32),
            pltpu.SemaphoreType.DMA,
            pltpu.SemaphoreType.DMA,
            pltpu.SemaphoreType.DMA,
            pltpu.SemaphoreType.DMA,
        ],
    )
    return kern(entity_embeddings, relation_embeddings, tidx, sidx, ridx)


def _gru_body(tails_ref, s_ref, r_ref, hl_ref, wt_ref, whh_ref, bih_ref,
              bhh_ref, out_ref, gib_ref):
    t = pl.program_id(1)
    H = out_ref.shape[1]
    f32 = jnp.float32
    bf16 = jnp.bfloat16

    @pl.when(t == 0)
    def _():
        s = s_ref[...].astype(bf16)
        r = r_ref[...].astype(bf16)
        gib_ref[...] = (
            jnp.dot(s, wt_ref[0:H, :], preferred_element_type=f32)
            + jnp.dot(r, wt_ref[H:2 * H, :], preferred_element_type=f32)
            + bih_ref[...]
        )
        out_ref[...] = jnp.zeros_like(out_ref)

    h = out_ref[...]
    x_t = tails_ref[0].astype(bf16)
    gi = gib_ref[...] + jnp.dot(x_t, wt_ref[2 * H:3 * H, :],
                                preferred_element_type=f32)
    gh = jnp.dot(h.astype(bf16), whh_ref[...],
                 preferred_element_type=f32) + bhh_ref[...]
    i_r, i_z, i_n = gi[:, :H], gi[:, H:2 * H], gi[:, 2 * H:]
    h_r, h_z, h_n = gh[:, :H], gh[:, H:2 * H], gh[:, 2 * H:]
    rg = jax.nn.sigmoid(i_r + h_r)
    z = jax.nn.sigmoid(i_z + h_z)
    n = jnp.tanh(i_n + rg * h_n)
    h_new = (1.0 - z) * n + z * h
    m = hl_ref[0] > t
    out_ref[...] = jnp.where(m, h_new, h)


def _gru(tails, s_rows, r_rows, hist_len, W_ih, W_hh, b_ih, b_hh, BB):
    T, B, H = tails.shape
    NB = B // BB
    wt = W_ih.T.astype(jnp.bfloat16)
    whh = W_hh.T.astype(jnp.bfloat16)
    bih = b_ih.reshape(1, 3 * H).astype(jnp.float32)
    bhh = b_hh.reshape(1, 3 * H).astype(jnp.float32)
    hl3 = hist_len.astype(jnp.int32).reshape(NB, BB, 1)

    grid = (NB, T)
    return pl.pallas_call(
        _gru_body,
        grid=grid,
        in_specs=[
            pl.BlockSpec((1, BB, H), lambda b, t: (t, b, 0)),
            pl.BlockSpec((BB, H), lambda b, t: (b, 0)),
            pl.BlockSpec((BB, H), lambda b, t: (b, 0)),
            pl.BlockSpec((1, BB, 1), lambda b, t: (b, 0, 0)),
            pl.BlockSpec((3 * H, 3 * H), lambda b, t: (0, 0)),
            pl.BlockSpec((H, 3 * H), lambda b, t: (0, 0)),
            pl.BlockSpec((1, 3 * H), lambda b, t: (0, 0)),
            pl.BlockSpec((1, 3 * H), lambda b, t: (0, 0)),
        ],
        out_specs=pl.BlockSpec((BB, H), lambda b, t: (b, 0)),
        out_shape=jax.ShapeDtypeStruct((B, H), jnp.float32),
        scratch_shapes=[pltpu.VMEM((BB, 3 * H), jnp.float32)],
        compiler_params=pltpu.CompilerParams(
            dimension_semantics=("arbitrary", "arbitrary"),
        ),
    )(tails, s_rows, r_rows, hl3, wt, whh, bih, bhh)


@jax.jit
def kernel(all_triples, hist_tails, hist_len, entity_embeddings,
           relation_embeddings, W_ih, W_hh, b_ih, b_hh):
    B, T = hist_tails.shape
    H = entity_embeddings.shape[1]

    NSPLIT = 2
    BC = B // NSPLIT
    outs = []
    for c in range(NSPLIT):
        sl = slice(c * BC, (c + 1) * BC)
        tidx = hist_tails[sl].T.astype(jnp.int32).reshape(
            _NW, (T * BC) // (_NW * _CHUNK), _CHUNK)
        sidx = all_triples[sl, 0].astype(jnp.int32).reshape(
            _NW, BC // (_NW * _CHUNK), _CHUNK)
        ridx = all_triples[sl, 1].astype(jnp.int32).reshape(
            _NW, BC // (_NW * _CHUNK), _CHUNK)
        tails_flat, s_rows, r_rows = _sc_gather(
            entity_embeddings, relation_embeddings, tidx, sidx, ridx, H)
        tails = tails_flat.reshape(T, BC, H)
        outs.append(_gru(tails, s_rows, r_rows, hist_len[sl],
                         W_ih, W_hh, b_ih, b_hh, BB=2048))
    return jnp.concatenate(outs, axis=0)


# speedup vs baseline: 19.4426x; 1.0499x over previous
import functools

import jax
import jax.numpy as jnp
from jax import lax
from jax.experimental import pallas as pl
from jax.experimental.pallas import tpu as pltpu
from jax.experimental.pallas import tpu_sc as plsc

_NC = 2
_NS = 16
_NW = _NC * _NS
_CHUNK = 128


def _stream_gather(table, idx_hbm, out_hbm, wid, nch, depth, idx_v, bufs,
                   gsems, wsems):
    pltpu.sync_copy(idx_hbm.at[wid], idx_v)
    base = wid * nch * _CHUNK

    def _gather(c, k):
        return pltpu.make_async_copy(table.at[idx_v.at[c]], bufs[k], gsems[k])

    def _wb(c, k):
        dst = out_hbm.at[pl.ds(base + c * _CHUNK, _CHUNK)]
        return pltpu.make_async_copy(bufs[k], dst, wsems[k])

    for k in range(depth):
        _gather(k, k).start()

    def outer(i, carry):
        for k in range(depth):
            c = i * depth + k
            _gather(c, k).wait()
            _wb(c, k).start()

            @pl.when(c + depth < nch)
            def _():
                _wb(c, k).wait()
                _gather(c + depth, k).start()

        return carry

    lax.fori_loop(0, nch // depth, outer, 0, unroll=False)

    for k in range(depth):
        _wb(nch - depth + k, k).wait()


def _sc_gather_body(ent_hbm, rel_hbm, tidx_hbm, sidx_hbm, ridx_hbm,
                    tails_out, s_out, r_out,
                    iv_t, iv_s, iv_r, buf0, buf1, buf2, buf3,
                    g0, g1, g2, g3, w0, w1, w2, w3):
    wid = lax.axis_index("s") * _NC + lax.axis_index("c")
    bufs = (buf0, buf1, buf2, buf3)
    gsems = (g0, g1, g2, g3)
    wsems = (w0, w1, w2, w3)
    nch_t = iv_t.shape[0]
    nch_s = iv_s.shape[0]
    nch_r = iv_r.shape[0]

    def depth_for(nch):
        for d in (4, 2, 1):
            if nch % d == 0 and nch >= d:
                return d
        return 1

    _stream_gather(ent_hbm, tidx_hbm, tails_out, wid, nch_t, depth_for(nch_t),
                   iv_t, bufs, gsems, wsems)
    _stream_gather(ent_hbm, sidx_hbm, s_out, wid, nch_s, depth_for(nch_s),
                   iv_s, bufs, gsems, wsems)
    _stream_gather(rel_hbm, ridx_hbm, r_out, wid, nch_r, depth_for(nch_r),
                   iv_r, bufs, gsems, wsems)


def _sc_gather(entity_embeddings, relation_embeddings, tidx, sidx, ridx, H):
    nch_t = tidx.shape[1]
    nch_s = sidx.shape[1]
    nch_r = ridx.shape[1]
    mesh = plsc.VectorSubcoreMesh(core_axis_name="c", subcore_axis_name="s",
                                  num_cores=_NC, num_subcores=_NS)
    f32 = jnp.float32
    kern = pl.kernel(
        _sc_gather_body,
        out_type=(
            jax.ShapeDtypeStruct((_NW * nch_t * _CHUNK, H), f32),
            jax.ShapeDtypeStruct((_NW * nch_s * _CHUNK, H), f32),
            jax.ShapeDtypeStruct((_NW * nch_r * _CHUNK, H), f32),
        ),
        mesh=mesh,
        scratch_types=[
            pltpu.VMEM((nch_t, _CHUNK), jnp.int32),
            pltpu.VMEM((nch_s, _CHUNK), jnp.int32),
            pltpu.VMEM((nch_r, _CHUNK), jnp.int32),
            pltpu.VMEM((_CHUNK, H), f32),
            pltpu.VMEM((_CHUNK, H), f32),
            pltpu.VMEM((_CHUNK, H), f32),
            pltpu.VMEM((_CHUNK, H), f32),
            pltpu.SemaphoreType.DMA,
            pltpu.SemaphoreType.DMA,
            pltpu.SemaphoreType.DMA,
            pltpu.SemaphoreType.DMA,
            pltpu.SemaphoreType.DMA,
            pltpu.SemaphoreType.DMA,
            pltpu.SemaphoreType.DMA,
            pltpu.SemaphoreType.DMA,
        ],
    )
    return kern(entity_embeddings, relation_embeddings, tidx, sidx, ridx)


def _gru_body(tails_ref, s_ref, r_ref, hl_ref, wt_ref, whh_ref, bih_ref,
              bhh_ref, out_ref, gib_ref):
    t = pl.program_id(1)
    H = out_ref.shape[1]
    f32 = jnp.float32
    bf16 = jnp.bfloat16

    @pl.when(t == 0)
    def _():
        s = s_ref[...].astype(bf16)
        r = r_ref[...].astype(bf16)
        gib_ref[...] = (
            jnp.dot(s, wt_ref[0:H, :], preferred_element_type=f32)
            + jnp.dot(r, wt_ref[H:2 * H, :], preferred_element_type=f32)
            + bih_ref[...]
        )
        out_ref[...] = jnp.zeros_like(out_ref)

    h = out_ref[...]
    x_t = tails_ref[0].astype(bf16)
    gi = gib_ref[...] + jnp.dot(x_t, wt_ref[2 * H:3 * H, :],
                                preferred_element_type=f32)
    gh = jnp.dot(h.astype(bf16), whh_ref[...],
                 preferred_element_type=f32) + bhh_ref[...]
    i_r, i_z, i_n = gi[:, :H], gi[:, H:2 * H], gi[:, 2 * H:]
    h_r, h_z, h_n = gh[:, :H], gh[:, H:2 * H], gh[:, 2 * H:]
    rg = 0.5 * jnp.tanh(0.5 * (i_r + h_r)) + 0.5
    z = 0.5 * jnp.tanh(0.5 * (i_z + h_z)) + 0.5
    n = jnp.tanh(i_n + rg * h_n)
    h_new = (1.0 - z) * n + z * h
    m = hl_ref[0] > t
    out_ref[...] = jnp.where(m, h_new, h)


def _gru(tails, s_rows, r_rows, hist_len, W_ih, W_hh, b_ih, b_hh, BB):
    T, B, H = tails.shape
    NB = B // BB
    wt = W_ih.T.astype(jnp.bfloat16)
    whh = W_hh.T.astype(jnp.bfloat16)
    bih = b_ih.reshape(1, 3 * H).astype(jnp.float32)
    bhh = b_hh.reshape(1, 3 * H).astype(jnp.float32)
    hl3 = hist_len.astype(jnp.int32).reshape(NB, BB, 1)

    grid = (NB, T)
    return pl.pallas_call(
        _gru_body,
        grid=grid,
        in_specs=[
            pl.BlockSpec((1, BB, H), lambda b, t: (t, b, 0)),
            pl.BlockSpec((BB, H), lambda b, t: (b, 0)),
            pl.BlockSpec((BB, H), lambda b, t: (b, 0)),
            pl.BlockSpec((1, BB, 1), lambda b, t: (b, 0, 0)),
            pl.BlockSpec((3 * H, 3 * H), lambda b, t: (0, 0)),
            pl.BlockSpec((H, 3 * H), lambda b, t: (0, 0)),
            pl.BlockSpec((1, 3 * H), lambda b, t: (0, 0)),
            pl.BlockSpec((1, 3 * H), lambda b, t: (0, 0)),
        ],
        out_specs=pl.BlockSpec((BB, H), lambda b, t: (b, 0)),
        out_shape=jax.ShapeDtypeStruct((B, H), jnp.float32),
        scratch_shapes=[pltpu.VMEM((BB, 3 * H), jnp.float32)],
        compiler_params=pltpu.CompilerParams(
            dimension_semantics=("arbitrary", "arbitrary"),
        ),
    )(tails, s_rows, r_rows, hl3, wt, whh, bih, bhh)


@jax.jit
def kernel(all_triples, hist_tails, hist_len, entity_embeddings,
           relation_embeddings, W_ih, W_hh, b_ih, b_hh):
    B, T = hist_tails.shape
    H = entity_embeddings.shape[1]

    NSPLIT = 2
    BC = B // NSPLIT
    outs = []
    for c in range(NSPLIT):
        sl = slice(c * BC, (c + 1) * BC)
        tidx = hist_tails[sl].T.astype(jnp.int32).reshape(
            _NW, (T * BC) // (_NW * _CHUNK), _CHUNK)
        sidx = all_triples[sl, 0].astype(jnp.int32).reshape(
            _NW, BC // (_NW * _CHUNK), _CHUNK)
        ridx = all_triples[sl, 1].astype(jnp.int32).reshape(
            _NW, BC // (_NW * _CHUNK), _CHUNK)
        tails_flat, s_rows, r_rows = _sc_gather(
            entity_embeddings, relation_embeddings, tidx, sidx, ridx, H)
        tails = tails_flat.reshape(T, BC, H)
        outs.append(_gru(tails, s_rows, r_rows, hist_len[sl],
                         W_ih, W_hh, b_ih, b_hh, BB=4096))
    return jnp.concatenate(outs, axis=0)
